# baseline (speedup 1.0000x reference)
import functools

import jax
import jax.numpy as jnp
from jax import lax
from jax.experimental import pallas as pl
from jax.experimental.pallas import tpu as pltpu
from jax.experimental.pallas import tpu_sc as plsc

B = 16384
L = 200
V = 5045
D = 50
VPAD = 5056
VREP = 16

NC, NS, LANES = 2, 16, 16
NW = NC * NS
ROWS_PER_W = B // NW

CHUNK = 64
NCHUNK = ROWS_PER_W // CHUNK
NBUF = 2


def _fold_head_body(table_ref, w_ref, b_ref, v_ref):
    t = table_ref[...]
    w = w_ref[...]
    s = jax.lax.dot_general(t, w, (((1,), (1,)), ((), ())),
                            preferred_element_type=jnp.float32)
    v = s * (1.0 / L) + b_ref[0] * (1.0 / L)
    v_ref[...] = jnp.broadcast_to(v, (VPAD, VREP))


def _fold_head(table, w, b):
    tpad = jnp.zeros((VPAD, D), jnp.float32).at[:V].set(table)
    v2d = pl.pallas_call(
        _fold_head_body,
        out_shape=jax.ShapeDtypeStruct((VPAD, VREP), jnp.float32),
        in_specs=[
            pl.BlockSpec(memory_space=pltpu.VMEM),
            pl.BlockSpec(memory_space=pltpu.VMEM),
            pl.BlockSpec(memory_space=pltpu.SMEM),
        ],
        out_specs=pl.BlockSpec(memory_space=pltpu.VMEM),
    )(tpad, w, b)
    return v2d.reshape(VPAD * VREP)


def _sc_body(x_hbm, v_hbm, out_hbm, x_v, v_v, o_v, sem_v, sem_x0, sem_x1):
    wid = lax.axis_index("s") * NC + lax.axis_index("c")
    row0 = wid * ROWS_PER_W
    sems = (sem_x0, sem_x1)

    cp_v = pltpu.make_async_copy(v_hbm, v_v, sem_v)
    cp_v.start()

    def x_copy(c, b):
        return pltpu.make_async_copy(
            x_hbm.at[pl.ds(row0 + c * CHUNK, CHUNK), :], x_v.at[b], sems[b])

    for b in range(NBUF):
        x_copy(b, b).start()
    cp_v.wait()

    lane = lax.iota(jnp.int32, LANES)
    lanec = lax.bitwise_and(lane, jnp.int32(VREP - 1))
    last_lane = lane == (LANES - 1)
    tailf = jnp.where(lane >= 8, jnp.float32(1.0), jnp.float32(0.0))

    col_starts = [16 * k for k in range(12)] + [184]

    vshift = VREP.bit_length() - 1

    def v4_gather(xi):
        return plsc.load_gather(
            v_v, [lax.bitwise_or(lax.shift_left(xi, vshift), lanec)])

    for c in range(NCHUNK):
        b = c % NBUF
        x_copy(c, b).wait()

        @pl.loop(0, CHUNK, unroll=4)
        def _row(r):
            parts = []
            for k, col in enumerate(col_starts):
                xi = x_v[b, r, pl.ds(col, LANES)]
                vals = v4_gather(xi)
                if k == len(col_starts) - 1:
                    vals = vals * tailf
                parts.append(vals)
            while len(parts) > 1:
                parts = [p0 + p1 for p0, p1 in
                         zip(parts[0::2], parts[1::2])] + (
                             [parts[-1]] if len(parts) % 2 else [])
            tot = plsc.cumsum(parts[0])
            oi = jnp.full((LANES,), c * CHUNK + r, jnp.int32)
            plsc.store_scatter(o_v, [oi], tot, mask=last_lane)

        if c + NBUF < NCHUNK:
            x_copy(c + NBUF, b).start()

    pltpu.sync_copy(o_v, out_hbm.at[pl.ds(row0, ROWS_PER_W)])


@functools.cache
def _sc_gather_sum():
    return pl.kernel(
        _sc_body,
        out_type=jax.ShapeDtypeStruct((B,), jnp.float32),
        mesh=plsc.VectorSubcoreMesh(core_axis_name="c", subcore_axis_name="s"),
        compiler_params=pltpu.CompilerParams(needs_layout_passes=False),
        scratch_types=[
            pltpu.VMEM((NBUF, CHUNK, L), jnp.int32),
            pltpu.VMEM((VPAD * VREP,), jnp.float32),
            pltpu.VMEM((ROWS_PER_W,), jnp.float32),
            pltpu.SemaphoreType.DMA,
            pltpu.SemaphoreType.DMA,
            pltpu.SemaphoreType.DMA,
        ],
    )


@jax.jit
def kernel(x, table, W, b):
    v = _fold_head(table.astype(jnp.float32), W.astype(jnp.float32),
                   b.astype(jnp.float32))
    out = _sc_gather_sum()(x.astype(jnp.int32), v)
    return out.reshape(B, 1)


# speedup vs baseline: 1.0744x; 1.0744x over previous
import functools

import jax
import jax.numpy as jnp
from jax import lax
from jax.experimental import pallas as pl
from jax.experimental.pallas import tpu as pltpu
from jax.experimental.pallas import tpu_sc as plsc

B = 16384
L = 200
V = 5045
D = 50
VPAD = 5120

NC, NS, LANES = 2, 16, 16
NW = NC * NS
ROWS_PER_W = B // NW
TROWS = VPAD // NS

CHUNK = 64
NCHUNK = ROWS_PER_W // CHUNK
NBUF = 2
RUNROLL = 4
DPAD = 64
WB = 80


def _sc_body(x_hbm, t_hbm, wb_hbm, out_hbm,
             x_v, v_v, o_v, t_v, wb_v, vsl_v, v_sp,
             sem_x0, sem_x1, sem_t, sem_w):
    cid = lax.axis_index("c")
    sid = lax.axis_index("s")
    wid = sid * NC + cid
    row0 = wid * ROWS_PER_W
    sems = (sem_x0, sem_x1)

    def x_copy(c, b):
        return pltpu.make_async_copy(
            x_hbm.at[pl.ds(row0 + c * CHUNK, CHUNK), :], x_v.at[b], sems[b])

    cp_t = pltpu.make_async_copy(t_hbm.at[pl.ds(sid * TROWS, TROWS), :],
                                 t_v, sem_t)
    cp_w = pltpu.make_async_copy(wb_hbm, wb_v, sem_w)
    cp_t.start()
    cp_w.start()
    for b in range(NBUF):
        x_copy(b, b).start()
    cp_t.wait()
    cp_w.wait()

    lane = lax.iota(jnp.int32, LANES)
    last_lane = lane == (LANES - 1)
    tailf = jnp.where(lane >= 8, jnp.float32(1.0), jnp.float32(0.0))

    col_starts = [16 * k for k in range(12)] + [184]

    inv_l = jnp.float32(1.0 / L)
    wv = [wb_v[pl.ds(16 * j, LANES)] for j in range(DPAD // LANES)]
    bias_v = wb_v[pl.ds(DPAD, LANES)] * inv_l

    @pl.loop(0, TROWS, step=RUNROLL)
    def _fold(t0):
        for u in range(RUNROLL):
            tr = t0 + u
            prods = [t_v[tr, pl.ds(16 * j, LANES)] * wv[j]
                     for j in range(DPAD // LANES)]
            tot = plsc.cumsum((prods[0] + prods[1]) + (prods[2] + prods[3]))
            vi = jnp.full((LANES,), tr, jnp.int32)
            plsc.store_scatter(vsl_v, [vi], tot * inv_l + bias_v,
                               mask=last_lane)

    pltpu.sync_copy(vsl_v, v_sp.at[pl.ds(sid * TROWS, TROWS)])
    plsc.subcore_barrier()
    pltpu.sync_copy(v_sp, v_v)

    @pl.loop(0, NCHUNK, step=NBUF)
    def _chunk(c0):
        for b in range(NBUF):
            c = c0 + b
            x_copy(c, b).wait()

            @pl.loop(0, CHUNK, step=RUNROLL)
            def _row(r0):
                for u in range(RUNROLL):
                    r = r0 + u
                    parts = []
                    for k, col in enumerate(col_starts):
                        xi = x_v[b, r, pl.ds(col, LANES)]
                        vals = plsc.load_gather(v_v, [xi])
                        if k == len(col_starts) - 1:
                            vals = vals * tailf
                        parts.append(vals)
                    while len(parts) > 1:
                        parts = [p0 + p1 for p0, p1 in
                                 zip(parts[0::2], parts[1::2])] + (
                                     [parts[-1]] if len(parts) % 2 else [])
                    tot = plsc.cumsum(parts[0])
                    oi = jnp.full((LANES,), c * CHUNK + r, jnp.int32)
                    plsc.store_scatter(o_v, [oi], tot, mask=last_lane)

            @pl.when(c + NBUF < NCHUNK)
            def _prefetch():
                x_copy(c + NBUF, b).start()

    pltpu.sync_copy(o_v, out_hbm.at[pl.ds(row0, ROWS_PER_W)])


@functools.cache
def _sc_kernel():
    return pl.kernel(
        _sc_body,
        out_type=jax.ShapeDtypeStruct((B,), jnp.float32),
        mesh=plsc.VectorSubcoreMesh(core_axis_name="c", subcore_axis_name="s"),
        compiler_params=pltpu.CompilerParams(needs_layout_passes=False),
        scratch_types=[
            pltpu.VMEM((NBUF, CHUNK, L), jnp.int32),
            pltpu.VMEM((VPAD,), jnp.float32),
            pltpu.VMEM((ROWS_PER_W,), jnp.float32),
            pltpu.VMEM((TROWS, DPAD), jnp.float32),
            pltpu.VMEM((WB,), jnp.float32),
            pltpu.VMEM((TROWS,), jnp.float32),
            pltpu.VMEM_SHARED((VPAD,), jnp.float32),
            pltpu.SemaphoreType.DMA,
            pltpu.SemaphoreType.DMA,
            pltpu.SemaphoreType.DMA,
            pltpu.SemaphoreType.DMA,
        ],
    )


@jax.jit
def kernel(x, table, W, b):
    tpad = jnp.zeros((VPAD, DPAD), jnp.float32).at[:V, :D].set(
        table.astype(jnp.float32))
    wb = jnp.zeros((WB,), jnp.float32).at[:D].set(
        W[0].astype(jnp.float32)).at[DPAD:].set(b[0].astype(jnp.float32))
    out = _sc_kernel()(x.astype(jnp.int32), tpad, wb)
    return out.reshape(B, 1)


# speedup vs baseline: 1.0951x; 1.0193x over previous
import functools

import jax
import jax.numpy as jnp
from jax import lax
from jax.experimental import pallas as pl
from jax.experimental.pallas import tpu as pltpu
from jax.experimental.pallas import tpu_sc as plsc

B = 16384
L = 200
V = 5045
D = 50
VPAD = 5056
VREP = 1

NC, NS, LANES = 2, 16, 16
NW = NC * NS
ROWS_PER_W = B // NW

CHUNK = 64
NCHUNK = ROWS_PER_W // CHUNK
NBUF = 2
RUNROLL = 4


def _fold_head_body(table_ref, w_ref, b_ref, v_ref):
    t = table_ref[...]
    w = w_ref[...]
    s = jax.lax.dot_general(t, w, (((1,), (1,)), ((), ())),
                            preferred_element_type=jnp.float32)
    v = s * (1.0 / L) + b_ref[0] * (1.0 / L)
    v_ref[...] = jnp.broadcast_to(v, (VPAD, VREP))


def _fold_head(table, w, b):
    tpad = jnp.zeros((VPAD, D), jnp.float32).at[:V].set(table)
    v2d = pl.pallas_call(
        _fold_head_body,
        out_shape=jax.ShapeDtypeStruct((VPAD, VREP), jnp.float32),
        in_specs=[
            pl.BlockSpec(memory_space=pltpu.VMEM),
            pl.BlockSpec(memory_space=pltpu.VMEM),
            pl.BlockSpec(memory_space=pltpu.SMEM),
        ],
        out_specs=pl.BlockSpec(memory_space=pltpu.VMEM),
    )(tpad, w, b)
    return v2d.reshape(VPAD * VREP)


def _sc_body(x_hbm, v_hbm, out_hbm, x_v, v_v, o_v, sem_v, sem_x0, sem_x1):
    wid = lax.axis_index("s") * NC + lax.axis_index("c")
    row0 = wid * ROWS_PER_W
    sems = (sem_x0, sem_x1)

    cp_v = pltpu.make_async_copy(v_hbm, v_v, sem_v)
    cp_v.start()

    def x_copy(c, b):
        return pltpu.make_async_copy(
            x_hbm.at[pl.ds(row0 + c * CHUNK, CHUNK), :], x_v.at[b], sems[b])

    for b in range(NBUF):
        x_copy(b, b).start()
    cp_v.wait()

    lane = lax.iota(jnp.int32, LANES)
    lanec = lax.bitwise_and(lane, jnp.int32(VREP - 1))
    last_lane = lane == (LANES - 1)
    tailf = jnp.where(lane >= 8, jnp.float32(1.0), jnp.float32(0.0))

    col_starts = [16 * k for k in range(12)] + [184]

    def v4_gather(xi):
        if VREP == 1:
            return plsc.load_gather(v_v, [xi])
        vshift = VREP.bit_length() - 1
        return plsc.load_gather(
            v_v, [lax.bitwise_or(lax.shift_left(xi, vshift), lanec)])

    @pl.loop(0, NCHUNK, step=NBUF)
    def _chunk(c0):
        for b in range(NBUF):
            c = c0 + b
            x_copy(c, b).wait()

            @pl.loop(0, CHUNK, step=RUNROLL)
            def _row(r0):
                for u in range(RUNROLL):
                    r = r0 + u
                    parts = []
                    for k, col in enumerate(col_starts):
                        xi = x_v[b, r, pl.ds(col, LANES)]
                        vals = v4_gather(xi)
                        if k == len(col_starts) - 1:
                            vals = vals * tailf
                        parts.append(vals)
                    while len(parts) > 1:
                        parts = [p0 + p1 for p0, p1 in
                                 zip(parts[0::2], parts[1::2])] + (
                                     [parts[-1]] if len(parts) % 2 else [])
                    tot = plsc.cumsum(parts[0])
                    oi = jnp.full((LANES,), c * CHUNK + r, jnp.int32)
                    plsc.store_scatter(o_v, [oi], tot, mask=last_lane)

            @pl.when(c + NBUF < NCHUNK)
            def _prefetch():
                x_copy(c + NBUF, b).start()

    pltpu.sync_copy(o_v, out_hbm.at[pl.ds(row0, ROWS_PER_W)])


@functools.cache
def _sc_gather_sum():
    return pl.kernel(
        _sc_body,
        out_type=jax.ShapeDtypeStruct((B,), jnp.float32),
        mesh=plsc.VectorSubcoreMesh(core_axis_name="c", subcore_axis_name="s"),
        compiler_params=pltpu.CompilerParams(needs_layout_passes=False),
        scratch_types=[
            pltpu.VMEM((NBUF, CHUNK, L), jnp.int32),
            pltpu.VMEM((VPAD * VREP,), jnp.float32),
            pltpu.VMEM((ROWS_PER_W,), jnp.float32),
            pltpu.SemaphoreType.DMA,
            pltpu.SemaphoreType.DMA,
            pltpu.SemaphoreType.DMA,
        ],
    )


@jax.jit
def kernel(x, table, W, b):
    v = _fold_head(table.astype(jnp.float32), W.astype(jnp.float32),
                   b.astype(jnp.float32))
    out = _sc_gather_sum()(x.astype(jnp.int32), v)
    return out.reshape(B, 1)
